# baseline (speedup 1.0000x reference)
import functools

import jax
import jax.numpy as jnp
from jax import lax
from jax.experimental import pallas as pl
from jax.experimental.pallas import tpu as pltpu
from jax.experimental.pallas import tpu_sc as plsc

N = 10000
E = 320000
D_NODE = 128
D_EDGE = 16
HALF = 64
D_OUT = 128

NC = 2
NS = 16
NW = NC * NS
PER_W = E // NW
CH = 100
CH_P = 104
NCH = PER_W // CH
NH = NCH // 2
CHX = 50
CHX_P = 56
NCHX = PER_W // CHX
NHX = NCHX // 4
PW_P = NCHX * CHX_P
N_ACC = 10112
ROWS_PER_TILE = N_ACC // NS
DUMMY = N

_SC_MESH = plsc.VectorSubcoreMesh(core_axis_name="c", subcore_axis_name="s")


@functools.partial(
    pl.kernel,
    out_type=jax.ShapeDtypeStruct((NC, N_ACC, D_NODE), jnp.float32),
    mesh=_SC_MESH,
    scratch_types=[
        pltpu.VMEM_SHARED((N_ACC, D_NODE), jnp.float32),
        pltpu.VMEM((PW_P,), jnp.int32),
        pltpu.VMEM((CHX_P,), jnp.int32),
        pltpu.VMEM((CHX_P,), jnp.int32),
        pltpu.VMEM((CHX_P,), jnp.int32),
        pltpu.VMEM((CHX_P,), jnp.int32),
        pltpu.VMEM((CHX_P, D_NODE), jnp.float32),
        pltpu.VMEM((CHX_P, D_NODE), jnp.float32),
        pltpu.VMEM((CHX_P, D_NODE), jnp.float32),
        pltpu.VMEM((CHX_P, D_NODE), jnp.float32),
        pltpu.SemaphoreType.DMA,
        pltpu.SemaphoreType.DMA,
        pltpu.SemaphoreType.DMA,
        pltpu.SemaphoreType.DMA,
        pltpu.SemaphoreType.DMA,
        pltpu.SemaphoreType.DMA,
        pltpu.SemaphoreType.DMA,
        pltpu.SemaphoreType.DMA,
    ],
)
def _sc_segsum_x(x_h, src_h, dst_h, zsx_h, sx_o,
                 sx_sh, src_f, d0, d1, d2, d3, b0, b1, b2, b3,
                 sd0, sd1, sd2, sd3, sb0, sb1, sb2, sb3):
    c = lax.axis_index("c")
    s = lax.axis_index("s")
    wid = c * NS + s
    r0 = s * ROWS_PER_TILE
    dbufs = (d0, d1, d2, d3)
    bufs = (b0, b1, b2, b3)
    dsems = (sd0, sd1, sd2, sd3)
    sems = (sb0, sb1, sb2, sb3)

    pltpu.sync_copy(zsx_h, sx_sh.at[pl.ds(r0, ROWS_PER_TILE)])
    pltpu.sync_copy(src_h.at[wid], src_f)
    plsc.subcore_barrier()

    def fetch(j, p):
        pltpu.async_copy(dst_h.at[wid, j], dbufs[p], dsems[p])
        pltpu.async_copy(x_h.at[src_f.at[pl.ds(j * CHX_P, CHX_P)]],
                         bufs[p], sems[p])

    def fetch_wait(p):
        pltpu.make_async_copy(dst_h.at[wid, 0], dbufs[p], dsems[p]).wait()
        pltpu.make_async_copy(x_h.at[src_f.at[pl.ds(0, CHX_P)]],
                              bufs[p], sems[p]).wait()

    for p in range(4):
        fetch(p, p)

    def step(k, carry):
        for p in range(4):
            fetch_wait(p)
            pltpu.sync_copy(bufs[p], sx_sh.at[dbufs[p]], add=True)

            @pl.when(k < NHX - 1)
            def _():
                fetch(4 * (k + 1) + p, p)
        return carry

    lax.fori_loop(0, NHX, step, 0)
    plsc.subcore_barrier()

    pltpu.sync_copy(sx_sh.at[pl.ds(r0, ROWS_PER_TILE)],
                    sx_o.at[c, pl.ds(r0, ROWS_PER_TILE)])


@functools.partial(
    pl.kernel,
    out_type=jax.ShapeDtypeStruct((NC, N_ACC, D_NODE), jnp.float32),
    mesh=_SC_MESH,
    scratch_types=[
        pltpu.VMEM_SHARED((N_ACC, D_NODE), jnp.float32),
        pltpu.VMEM((CH_P,), jnp.int32),
        pltpu.VMEM((CH_P,), jnp.int32),
        pltpu.VMEM((CH_P, D_NODE), jnp.float32),
        pltpu.VMEM((CH_P, D_NODE), jnp.float32),
        pltpu.SemaphoreType.DMA,
        pltpu.SemaphoreType.DMA,
        pltpu.SemaphoreType.DMA,
        pltpu.SemaphoreType.DMA,
    ],
)
def _sc_segsum_ea(ea_h, dst_h, zse_h, se_o,
                  se_sh, dst_a, dst_b, ea_a, ea_b,
                  sem_a, sem_b, semd_a, semd_b):
    c = lax.axis_index("c")
    s = lax.axis_index("s")
    wid = c * NS + s
    r0 = s * ROWS_PER_TILE

    pltpu.sync_copy(zse_h, se_sh.at[pl.ds(r0, ROWS_PER_TILE)])
    plsc.subcore_barrier()

    def fetch(j, dbuf, dsem, buf, sem):
        pltpu.async_copy(dst_h.at[wid, j], dbuf, dsem)
        pltpu.async_copy(ea_h.at[wid, j], buf, sem)

    def fetch_wait(dbuf, dsem, buf, sem):
        pltpu.make_async_copy(dst_h.at[wid, 0], dbuf, dsem).wait()
        pltpu.make_async_copy(ea_h.at[wid, 0], buf, sem).wait()

    fetch(0, dst_a, semd_a, ea_a, sem_a)

    def step(k, carry):
        j0 = 2 * k
        j1 = j0 + 1
        fetch(j1, dst_b, semd_b, ea_b, sem_b)
        fetch_wait(dst_a, semd_a, ea_a, sem_a)
        pltpu.sync_copy(ea_a, se_sh.at[dst_a], add=True)

        @pl.when(k < NH - 1)
        def _():
            fetch(j0 + 2, dst_a, semd_a, ea_a, sem_a)

        fetch_wait(dst_b, semd_b, ea_b, sem_b)
        pltpu.sync_copy(ea_b, se_sh.at[dst_b], add=True)
        return carry

    lax.fori_loop(0, NH, step, 0)
    plsc.subcore_barrier()

    pltpu.sync_copy(se_sh.at[pl.ds(r0, ROWS_PER_TILE)],
                    se_o.at[c, pl.ds(r0, ROWS_PER_TILE)])


ROWS_B = 1000


def _tc_body(x_ref, sxp_ref, sep_ref,
             wn_ref, bn_ref, we_ref, be_ref, wr_ref, br_ref, z_ref):
    x = x_ref[...]
    sx = sxp_ref[0] + sxp_ref[1]
    sea = sep_ref[0] + sep_ref[1]
    se = sea[:, :D_EDGE]
    deg = sea[:, D_EDGE:D_EDGE + 1]
    inv = 1.0 / (deg + 1.0)
    h1 = jnp.dot((x + sx) * inv, wn_ref[...],
                 preferred_element_type=jnp.float32) + bn_ref[...]
    h2 = jnp.dot(se * inv, we_ref[...],
                 preferred_element_type=jnp.float32) + (deg * inv) * be_ref[...]
    h = jnp.concatenate([h1, h2], axis=1)
    z_ref[...] = jnp.dot(h, wr_ref[...],
                         preferred_element_type=jnp.float32) + br_ref[...]


def _tc_dense(x, sxp, sep, Wn, bn, We, be, Wr, br):
    grid = (N // ROWS_B,)
    full = lambda shape: pl.BlockSpec(shape, lambda i: (0,) * len(shape))
    return pl.pallas_call(
        _tc_body,
        grid=grid,
        in_specs=[
            pl.BlockSpec((ROWS_B, D_NODE), lambda i: (i, 0)),
            pl.BlockSpec((NC, ROWS_B, D_NODE), lambda i: (0, i, 0)),
            pl.BlockSpec((NC, ROWS_B, D_NODE), lambda i: (0, i, 0)),
            full((D_NODE, HALF)),
            full((1, HALF)),
            full((D_EDGE, HALF)),
            full((1, HALF)),
            full((2 * HALF, D_OUT)),
            full((1, D_OUT)),
        ],
        out_specs=pl.BlockSpec((ROWS_B, D_OUT), lambda i: (i, 0)),
        out_shape=jax.ShapeDtypeStruct((N, D_OUT), jnp.float32),
    )(x, sxp, sep, Wn, bn, We, be, Wr, br)


def kernel(x, edge_index, edge_attr, Wn, bn, We, be, Wr, br):
    src2 = jnp.pad(edge_index[0].reshape(NW, NCHX, CHX),
                   ((0, 0), (0, 0), (0, CHX_P - CHX))).reshape(NW, PW_P)
    dst4 = jnp.pad(edge_index[1].reshape(NW, NCHX, CHX),
                   ((0, 0), (0, 0), (0, CHX_P - CHX)),
                   constant_values=DUMMY)
    dst3 = jnp.pad(edge_index[1].reshape(NW, NCH, CH),
                   ((0, 0), (0, 0), (0, CH_P - CH)),
                   constant_values=DUMMY)
    ea_wide = jnp.concatenate(
        [edge_attr, jnp.ones((E, 1), jnp.float32),
         jnp.zeros((E, D_NODE - D_EDGE - 1), jnp.float32)], axis=1)
    ea4 = jnp.pad(ea_wide.reshape(NW, NCH, CH, D_NODE),
                  ((0, 0), (0, 0), (0, CH_P - CH), (0, 0)))
    zsx = jnp.zeros((ROWS_PER_TILE, D_NODE), jnp.float32)
    zse = jnp.zeros((ROWS_PER_TILE, D_NODE), jnp.float32)
    sxp = _sc_segsum_x(x, src2, dst4, zsx)
    sep = _sc_segsum_ea(ea4, dst3, zse)
    return _tc_dense(x, sxp, sep, Wn, bn.reshape(1, HALF), We,
                     be.reshape(1, HALF), Wr, br.reshape(1, D_OUT))


# speedup vs baseline: 2.3128x; 2.3128x over previous
import functools

import jax
import jax.numpy as jnp
from jax import lax
from jax.experimental import pallas as pl
from jax.experimental.pallas import tpu as pltpu
from jax.experimental.pallas import tpu_sc as plsc

N = 10000
E = 320000
D_NODE = 128
D_EDGE = 16
HALF = 64
D_OUT = 128

NC = 2
NS = 16
NW = NC * NS
PER_W = E // NW
CH = 100
CH_P = 104
NCH = PER_W // CH
NH = NCH // 2
CHX = 250
CHX_P = 256
NCHX = PER_W // CHX
PW_P = NCHX * CHX_P
N_ACC = 10112
ROWS_PER_TILE = N_ACC // NS
DUMMY = N

_SC_MESH = plsc.VectorSubcoreMesh(core_axis_name="c", subcore_axis_name="s")


@functools.partial(
    pl.kernel,
    out_type=jax.ShapeDtypeStruct((NC, N_ACC, D_NODE), jnp.float32),
    mesh=_SC_MESH,
    scratch_types=[
        pltpu.VMEM_SHARED((N_ACC, D_NODE), jnp.float32),
        pltpu.VMEM((PW_P,), jnp.int32),
        pltpu.VMEM((CHX_P,), jnp.int32),
        pltpu.VMEM((CHX_P, D_NODE), jnp.float32),
        pltpu.SemaphoreType.DMA,
    ],
)
def _sc_segsum_x(x_h, src_h, dst_h, zsx_h, sx_o,
                 sx_sh, src_f, drow_v, rows_v, sem):
    c = lax.axis_index("c")
    s = lax.axis_index("s")
    wid = c * NS + s
    r0 = s * ROWS_PER_TILE

    pltpu.sync_copy(zsx_h, sx_sh.at[pl.ds(r0, ROWS_PER_TILE)])
    pltpu.sync_copy(src_h.at[wid], src_f)
    plsc.subcore_barrier()

    def step(j, carry):
        pltpu.sync_copy(dst_h.at[wid, j], drow_v)
        pltpu.async_copy(x_h.at[src_f.at[pl.ds(j * CHX_P, CHX_P)]],
                         rows_v, sem).wait()
        pltpu.sync_copy(rows_v, sx_sh.at[drow_v], add=True)
        return carry

    lax.fori_loop(0, NCHX, step, 0)
    plsc.subcore_barrier()

    pltpu.sync_copy(sx_sh.at[pl.ds(r0, ROWS_PER_TILE)],
                    sx_o.at[c, pl.ds(r0, ROWS_PER_TILE)])


@functools.partial(
    pl.kernel,
    out_type=jax.ShapeDtypeStruct((NC, N_ACC, D_NODE), jnp.float32),
    mesh=_SC_MESH,
    scratch_types=[
        pltpu.VMEM_SHARED((N_ACC, D_NODE), jnp.float32),
        pltpu.VMEM((CH_P,), jnp.int32),
        pltpu.VMEM((CH_P,), jnp.int32),
        pltpu.VMEM((CH_P, D_NODE), jnp.float32),
        pltpu.VMEM((CH_P, D_NODE), jnp.float32),
        pltpu.SemaphoreType.DMA,
        pltpu.SemaphoreType.DMA,
        pltpu.SemaphoreType.DMA,
        pltpu.SemaphoreType.DMA,
    ],
)
def _sc_segsum_ea(ea_h, dst_h, zse_h, se_o,
                  se_sh, dst_a, dst_b, ea_a, ea_b,
                  sem_a, sem_b, semd_a, semd_b):
    c = lax.axis_index("c")
    s = lax.axis_index("s")
    wid = c * NS + s
    r0 = s * ROWS_PER_TILE

    pltpu.sync_copy(zse_h, se_sh.at[pl.ds(r0, ROWS_PER_TILE)])
    plsc.subcore_barrier()

    def fetch(j, dbuf, dsem, buf, sem):
        pltpu.async_copy(dst_h.at[wid, j], dbuf, dsem)
        pltpu.async_copy(ea_h.at[wid, j], buf, sem)

    def fetch_wait(dbuf, dsem, buf, sem):
        pltpu.make_async_copy(dst_h.at[wid, 0], dbuf, dsem).wait()
        pltpu.make_async_copy(ea_h.at[wid, 0], buf, sem).wait()

    fetch(0, dst_a, semd_a, ea_a, sem_a)

    def step(k, carry):
        j0 = 2 * k
        j1 = j0 + 1
        fetch(j1, dst_b, semd_b, ea_b, sem_b)
        fetch_wait(dst_a, semd_a, ea_a, sem_a)
        pltpu.sync_copy(ea_a, se_sh.at[dst_a], add=True)

        @pl.when(k < NH - 1)
        def _():
            fetch(j0 + 2, dst_a, semd_a, ea_a, sem_a)

        fetch_wait(dst_b, semd_b, ea_b, sem_b)
        pltpu.sync_copy(ea_b, se_sh.at[dst_b], add=True)
        return carry

    lax.fori_loop(0, NH, step, 0)
    plsc.subcore_barrier()

    pltpu.sync_copy(se_sh.at[pl.ds(r0, ROWS_PER_TILE)],
                    se_o.at[c, pl.ds(r0, ROWS_PER_TILE)])


ROWS_B = 1000


def _tc_body(x_ref, sxp_ref, sep_ref,
             wn_ref, bn_ref, we_ref, be_ref, wr_ref, br_ref, z_ref):
    x = x_ref[...]
    sx = sxp_ref[0] + sxp_ref[1]
    sea = sep_ref[0] + sep_ref[1]
    se = sea[:, :D_EDGE]
    deg = sea[:, D_EDGE:D_EDGE + 1]
    inv = 1.0 / (deg + 1.0)
    h1 = jnp.dot((x + sx) * inv, wn_ref[...],
                 preferred_element_type=jnp.float32) + bn_ref[...]
    h2 = jnp.dot(se * inv, we_ref[...],
                 preferred_element_type=jnp.float32) + (deg * inv) * be_ref[...]
    h = jnp.concatenate([h1, h2], axis=1)
    z_ref[...] = jnp.dot(h, wr_ref[...],
                         preferred_element_type=jnp.float32) + br_ref[...]


def _tc_dense(x, sxp, sep, Wn, bn, We, be, Wr, br):
    grid = (N // ROWS_B,)
    full = lambda shape: pl.BlockSpec(shape, lambda i: (0,) * len(shape))
    return pl.pallas_call(
        _tc_body,
        grid=grid,
        in_specs=[
            pl.BlockSpec((ROWS_B, D_NODE), lambda i: (i, 0)),
            pl.BlockSpec((NC, ROWS_B, D_NODE), lambda i: (0, i, 0)),
            pl.BlockSpec((NC, ROWS_B, D_NODE), lambda i: (0, i, 0)),
            full((D_NODE, HALF)),
            full((1, HALF)),
            full((D_EDGE, HALF)),
            full((1, HALF)),
            full((2 * HALF, D_OUT)),
            full((1, D_OUT)),
        ],
        out_specs=pl.BlockSpec((ROWS_B, D_OUT), lambda i: (i, 0)),
        out_shape=jax.ShapeDtypeStruct((N, D_OUT), jnp.float32),
    )(x, sxp, sep, Wn, bn, We, be, Wr, br)


def kernel(x, edge_index, edge_attr, Wn, bn, We, be, Wr, br):
    src2 = jnp.pad(edge_index[0].reshape(NW, NCHX, CHX),
                   ((0, 0), (0, 0), (0, CHX_P - CHX))).reshape(NW, PW_P)
    dst4 = jnp.pad(edge_index[1].reshape(NW, NCHX, CHX),
                   ((0, 0), (0, 0), (0, CHX_P - CHX)),
                   constant_values=DUMMY)
    dst3 = jnp.pad(edge_index[1].reshape(NW, NCH, CH),
                   ((0, 0), (0, 0), (0, CH_P - CH)),
                   constant_values=DUMMY)
    ea_wide = jnp.concatenate(
        [edge_attr, jnp.ones((E, 1), jnp.float32),
         jnp.zeros((E, D_NODE - D_EDGE - 1), jnp.float32)], axis=1)
    ea4 = jnp.pad(ea_wide.reshape(NW, NCH, CH, D_NODE),
                  ((0, 0), (0, 0), (0, CH_P - CH), (0, 0)))
    zsx = jnp.zeros((ROWS_PER_TILE, D_NODE), jnp.float32)
    zse = jnp.zeros((ROWS_PER_TILE, D_NODE), jnp.float32)
    sxp = _sc_segsum_x(x, src2, dst4, zsx)
    sep = _sc_segsum_ea(ea4, dst3, zse)
    return _tc_dense(x, sxp, sep, Wn, bn.reshape(1, HALF), We,
                     be.reshape(1, HALF), Wr, br.reshape(1, D_OUT))
